# baseline (speedup 1.0000x reference)
import functools

import jax
import jax.numpy as jnp
from jax.experimental import pallas as pl
from jax.experimental.pallas import tpu as pltpu
from jax.experimental.pallas import tpu_sc as plsc


BN = 256


def _lex_sel(v1, i1, v2, i2):
    pred = (v2 > v1) | ((v2 == v1) & (i2 < i1))
    return jnp.where(pred, v2, v1), jnp.where(pred, i2, i1)


def _dist_argmax_kernel(x_ref, e_ref, dist_ref, ind_ref):
    dist_ref[...] = jax.lax.dot_general(
        x_ref[...], e_ref[...], (((1,), (1,)), ((), ())),
        preferred_element_type=jnp.float32)

    blockt = jax.lax.dot_general(
        e_ref[...], x_ref[...], (((1,), (1,)), ((), ())),
        preferred_element_type=jnp.float32)

    c = blockt.shape[0]
    iota8 = jax.lax.broadcasted_iota(jnp.int32, (8, BN), 0)
    cur = blockt[0:8]
    curi = iota8
    for r in range(1, c // 8):
        nxt = blockt[8 * r:8 * (r + 1)]
        pred = nxt > cur
        cur = jnp.where(pred, nxt, cur)
        curi = jnp.where(pred, iota8 + 8 * r, curi)

    v, i = _lex_sel(cur[0:4], curi[0:4], cur[4:8], curi[4:8])
    v, i = _lex_sel(v[0:2], i[0:2], v[2:4], i[2:4])
    v, i = _lex_sel(v[0:1], i[0:1], v[1:2], i[1:2])
    ind_ref[...] = i[None]


def _dist_argmax(flat_x, embed2d):
    n, d = flat_x.shape
    c = embed2d.shape[0]
    dist, ind = pl.pallas_call(
        _dist_argmax_kernel,
        grid=(n // BN,),
        in_specs=[
            pl.BlockSpec((BN, d), lambda i: (i, 0)),
            pl.BlockSpec((c, d), lambda i: (0, 0)),
        ],
        out_specs=[
            pl.BlockSpec((BN, c), lambda i: (i, 0)),
            pl.BlockSpec((1, 1, BN), lambda i: (i, 0, 0)),
        ],
        out_shape=[
            jax.ShapeDtypeStruct((n, c), jnp.float32),
            jax.ShapeDtypeStruct((n // BN, 1, BN), jnp.int32),
        ],
        compiler_params=pltpu.CompilerParams(
            dimension_semantics=("arbitrary",)),
    )(flat_x, embed2d)
    return dist, ind.reshape(n)


def _sc_gather(table, idx):
    info = plsc.get_sparse_core_info()
    nw = info.num_cores * info.num_subcores
    b = idx.shape[0]
    d_dim = table.shape[1]
    b_per_w = b // nw
    mesh = plsc.VectorSubcoreMesh(core_axis_name="c", subcore_axis_name="s")

    @functools.partial(
        pl.kernel, mesh=mesh,
        out_type=jax.ShapeDtypeStruct((b, d_dim), jnp.float32),
        scratch_types=[
            pltpu.VMEM((b_per_w,), jnp.int32),
            pltpu.VMEM((b_per_w, d_dim), jnp.float32),
            pltpu.SemaphoreType.DMA,
        ],
    )
    def k(table_hbm, idx_hbm, out_hbm, idx_v, rows_v, sem):
        wid = jax.lax.axis_index("s") * info.num_cores + jax.lax.axis_index("c")
        base = wid * b_per_w
        pltpu.sync_copy(idx_hbm.at[pl.ds(base, b_per_w)], idx_v)
        pltpu.async_copy(table_hbm.at[idx_v], rows_v, sem).wait()
        pltpu.sync_copy(rows_v, out_hbm.at[pl.ds(base, b_per_w)])

    return k(table, idx)


def kernel(x, embed):
    x = x.astype(jnp.float32)
    b, n, d = x.shape
    e2 = embed[0]
    flat = x.reshape(b * n, d)
    dist, ind = _dist_argmax(flat, e2)
    quantize = _sc_gather(e2, ind).reshape(b, n, d)
    return (quantize, ind.reshape(b, n), dist.reshape(b, n, -1))


# speedup vs baseline: 1.1120x; 1.1120x over previous
import functools

import jax
import jax.numpy as jnp
from jax.experimental import pallas as pl
from jax.experimental.pallas import tpu as pltpu
from jax.experimental.pallas import tpu_sc as plsc


BN = 512


def _lex_sel(v1, i1, v2, i2):
    pred = (v2 > v1) | ((v2 == v1) & (i2 < i1))
    return jnp.where(pred, v2, v1), jnp.where(pred, i2, i1)


def _dist_argmax_kernel(x_ref, e_ref, dist_ref, ind_ref):
    dist_ref[...] = jax.lax.dot_general(
        x_ref[...], e_ref[...], (((1,), (1,)), ((), ())),
        preferred_element_type=jnp.float32)

    blockt = jax.lax.dot_general(
        e_ref[...], x_ref[...], (((1,), (1,)), ((), ())),
        preferred_element_type=jnp.float32)

    c = blockt.shape[0]
    iota8 = jax.lax.broadcasted_iota(jnp.int32, (8, BN), 0)
    cur = blockt[0:8]
    curi = iota8
    for r in range(1, c // 8):
        nxt = blockt[8 * r:8 * (r + 1)]
        pred = nxt > cur
        cur = jnp.where(pred, nxt, cur)
        curi = jnp.where(pred, iota8 + 8 * r, curi)

    v, i = _lex_sel(cur[0:4], curi[0:4], cur[4:8], curi[4:8])
    v, i = _lex_sel(v[0:2], i[0:2], v[2:4], i[2:4])
    v, i = _lex_sel(v[0:1], i[0:1], v[1:2], i[1:2])
    ind_ref[...] = i[None]


def _dist_argmax(flat_x, embed2d):
    n, d = flat_x.shape
    c = embed2d.shape[0]
    dist, ind = pl.pallas_call(
        _dist_argmax_kernel,
        grid=(n // BN,),
        in_specs=[
            pl.BlockSpec((BN, d), lambda i: (i, 0)),
            pl.BlockSpec((c, d), lambda i: (0, 0)),
        ],
        out_specs=[
            pl.BlockSpec((BN, c), lambda i: (i, 0)),
            pl.BlockSpec((1, 1, BN), lambda i: (i, 0, 0)),
        ],
        out_shape=[
            jax.ShapeDtypeStruct((n, c), jnp.float32),
            jax.ShapeDtypeStruct((n // BN, 1, BN), jnp.int32),
        ],
        compiler_params=pltpu.CompilerParams(
            dimension_semantics=("arbitrary",)),
    )(flat_x, embed2d)
    return dist, ind.reshape(n)


def _sc_gather(table, idx):
    info = plsc.get_sparse_core_info()
    nw = info.num_cores * info.num_subcores
    b = idx.shape[0]
    d_dim = table.shape[1]
    b_per_w = b // nw
    mesh = plsc.VectorSubcoreMesh(core_axis_name="c", subcore_axis_name="s")

    @functools.partial(
        pl.kernel, mesh=mesh,
        out_type=jax.ShapeDtypeStruct((b, d_dim), jnp.float32),
        scratch_types=[
            pltpu.VMEM((b_per_w,), jnp.int32),
            pltpu.VMEM((b_per_w, d_dim), jnp.float32),
            pltpu.SemaphoreType.DMA,
        ],
    )
    def k(table_hbm, idx_hbm, out_hbm, idx_v, rows_v, sem):
        wid = jax.lax.axis_index("s") * info.num_cores + jax.lax.axis_index("c")
        base = wid * b_per_w
        pltpu.sync_copy(idx_hbm.at[pl.ds(base, b_per_w)], idx_v)
        pltpu.async_copy(table_hbm.at[idx_v], rows_v, sem).wait()
        pltpu.sync_copy(rows_v, out_hbm.at[pl.ds(base, b_per_w)])

    return k(table, idx)


def kernel(x, embed):
    x = x.astype(jnp.float32)
    b, n, d = x.shape
    e2 = embed[0]
    flat = x.reshape(b * n, d)
    dist, ind = _dist_argmax(flat, e2)
    quantize = _sc_gather(e2, ind).reshape(b, n, d)
    return (quantize, ind.reshape(b, n), dist.reshape(b, n, -1))
